# baseline (speedup 1.0000x reference)
import jax
import jax.numpy as jnp
from jax import lax
from jax.experimental import pallas as pl
from jax.experimental.pallas import tpu as pltpu

_B = 1024
_NI = 512
_NO = 64
_H = 128

_NCH = 16
_OC = _NO // _NCH


def _body(l1_hbm, l2_hbm, l3_hbm, state_hbm, opt_hbm, out_ref,
          l1_v, l2_v, l3_v, state_v, opt_v, v_s, sems):
    cp_l2 = pltpu.make_async_copy(l2_hbm, l2_v, sems.at[_NCH])
    cp_l3 = pltpu.make_async_copy(l3_hbm, l3_v, sems.at[_NCH + 1])
    cp_st = pltpu.make_async_copy(state_hbm, state_v, sems.at[_NCH + 2])
    cp_opt = pltpu.make_async_copy(opt_hbm, opt_v, sems.at[_NCH + 3])
    cp_l2.start()
    cp_l3.start()
    cp_st.start()
    cp_opt.start()
    cps = []
    for k in range(_NCH):
        cp = pltpu.make_async_copy(l1_hbm.at[pl.ds(k * _OC, _OC)],
                                   l1_v.at[pl.ds(k * _OC, _OC)],
                                   sems.at[k])
        cp.start()
        cps.append(cp)

    cp_l2.wait()
    cp_l3.wait()
    u = lax.dot_general(l3_v[...], l2_v[...], (((1,), (2,)), ((0,), (0,))),
                        preferred_element_type=jnp.float32)

    for k in range(_NCH):
        cps[k].wait()
        l1b = l1_v[pl.ds(k * _OC, _OC)]
        uk = u[k * _OC:(k + 1) * _OC]
        vrow = lax.dot_general(uk, l1b, (((2,), (2,)), ((0,), (0,))),
                               preferred_element_type=jnp.float32)
        v_s[pl.ds(k * _OC, _OC), :] = vrow.reshape(_OC, _NI)

    cp_st.wait()
    cp_opt.wait()
    scores = lax.dot_general(state_v[...], v_s[...], (((1,), (1,)), ((), ())),
                             preferred_element_type=jnp.float32)
    onehot = (opt_v[...] == lax.broadcasted_iota(jnp.int32, (1, _NO), 1))
    out_ref[...] = jnp.sum(jnp.where(onehot, scores, 0.0), axis=1,
                           keepdims=True)


def kernel(state, option, action, linear1, linear2, linear3):
    opt = option.astype(jnp.int32).reshape(_B, 1)
    hbm = pl.BlockSpec(memory_space=pltpu.MemorySpace.HBM)
    out = pl.pallas_call(
        _body,
        in_specs=[hbm, hbm, hbm, hbm, hbm],
        out_specs=pl.BlockSpec(memory_space=pltpu.MemorySpace.VMEM),
        out_shape=jax.ShapeDtypeStruct((_B, 1), jnp.float32),
        scratch_shapes=[
            pltpu.VMEM((_NO, _NI, _H), jnp.float32),
            pltpu.VMEM((_NO, _H, _H), jnp.float32),
            pltpu.VMEM((_NO, _H, 1), jnp.float32),
            pltpu.VMEM((_B, _NI), jnp.float32),
            pltpu.VMEM((_B, 1), jnp.int32),
            pltpu.VMEM((_NO, _NI), jnp.float32),
            pltpu.SemaphoreType.DMA((_NCH + 4,)),
        ],
    )(linear1, linear2, linear3, state, opt)
    return out


# speedup vs baseline: 1.0065x; 1.0065x over previous
import jax
import jax.numpy as jnp
from jax import lax
from jax.experimental import pallas as pl
from jax.experimental.pallas import tpu as pltpu

_B = 1024
_NI = 512
_NO = 64
_H = 128

_NCH = 4
_OC = _NO // _NCH


def _body(l1_hbm, l2_hbm, l3_hbm, state_hbm, opt_hbm, out_ref,
          l1_v, l2_v, l3_v, state_v, opt_v, v_s, sems):
    cp_l2 = pltpu.make_async_copy(l2_hbm, l2_v, sems.at[_NCH])
    cp_l3 = pltpu.make_async_copy(l3_hbm, l3_v, sems.at[_NCH + 1])
    cp_st = pltpu.make_async_copy(state_hbm, state_v, sems.at[_NCH + 2])
    cp_opt = pltpu.make_async_copy(opt_hbm, opt_v, sems.at[_NCH + 3])
    cp_l2.start()
    cp_l3.start()
    cp_st.start()
    cp_opt.start()
    cps = []
    for k in range(_NCH):
        cp = pltpu.make_async_copy(l1_hbm.at[pl.ds(k * _OC, _OC)],
                                   l1_v.at[pl.ds(k * _OC, _OC)],
                                   sems.at[k])
        cp.start()
        cps.append(cp)

    cp_l2.wait()
    cp_l3.wait()
    u = lax.dot_general(l3_v[...], l2_v[...], (((1,), (2,)), ((0,), (0,))),
                        preferred_element_type=jnp.float32)

    for k in range(_NCH):
        cps[k].wait()
        l1b = l1_v[pl.ds(k * _OC, _OC)]
        uk = u[k * _OC:(k + 1) * _OC]
        vrow = lax.dot_general(uk, l1b, (((2,), (2,)), ((0,), (0,))),
                               preferred_element_type=jnp.float32)
        v_s[pl.ds(k * _OC, _OC), :] = vrow.reshape(_OC, _NI)

    cp_st.wait()
    cp_opt.wait()
    scores = lax.dot_general(state_v[...], v_s[...], (((1,), (1,)), ((), ())),
                             preferred_element_type=jnp.float32)
    onehot = (opt_v[...] == lax.broadcasted_iota(jnp.int32, (1, _NO), 1))
    out_ref[...] = jnp.sum(jnp.where(onehot, scores, 0.0), axis=1,
                           keepdims=True)


def kernel(state, option, action, linear1, linear2, linear3):
    opt = option.astype(jnp.int32).reshape(_B, 1)
    hbm = pl.BlockSpec(memory_space=pltpu.MemorySpace.HBM)
    out = pl.pallas_call(
        _body,
        in_specs=[hbm, hbm, hbm, hbm, hbm],
        out_specs=pl.BlockSpec(memory_space=pltpu.MemorySpace.VMEM),
        out_shape=jax.ShapeDtypeStruct((_B, 1), jnp.float32),
        scratch_shapes=[
            pltpu.VMEM((_NO, _NI, _H), jnp.float32),
            pltpu.VMEM((_NO, _H, _H), jnp.float32),
            pltpu.VMEM((_NO, _H, 1), jnp.float32),
            pltpu.VMEM((_B, _NI), jnp.float32),
            pltpu.VMEM((_B, 1), jnp.int32),
            pltpu.VMEM((_NO, _NI), jnp.float32),
            pltpu.SemaphoreType.DMA((_NCH + 4,)),
        ],
    )(linear1, linear2, linear3, state, opt)
    return out


# speedup vs baseline: 1.0205x; 1.0139x over previous
import jax
import jax.numpy as jnp
from jax import lax
from jax.experimental import pallas as pl
from jax.experimental.pallas import tpu as pltpu

_B = 1024
_NI = 512
_NO = 64
_H = 128

_NCH = 8
_OC = _NO // _NCH


def _body(l1_hbm, l2_hbm, l3_hbm, state_hbm, opt_hbm, out_ref,
          l1_v, l2_v, l3_v, state_v, opt_v, v_s, sems):
    cp_l2 = pltpu.make_async_copy(l2_hbm, l2_v, sems.at[_NCH])
    cp_l3 = pltpu.make_async_copy(l3_hbm, l3_v, sems.at[_NCH + 1])
    cp_st = pltpu.make_async_copy(state_hbm, state_v, sems.at[_NCH + 2])
    cp_opt = pltpu.make_async_copy(opt_hbm, opt_v, sems.at[_NCH + 3])
    cp_l2.start()
    cp_l3.start()
    cp_st.start()
    cp_opt.start()
    cps = []
    for k in range(_NCH):
        cp = pltpu.make_async_copy(l1_hbm.at[pl.ds(k * _OC, _OC)],
                                   l1_v.at[pl.ds(k * _OC, _OC)],
                                   sems.at[k])
        cp.start()
        cps.append(cp)

    cp_l2.wait()
    cp_l3.wait()
    u = lax.dot_general(l3_v[...], l2_v[...], (((1,), (2,)), ((0,), (0,))),
                        preferred_element_type=jnp.float32)

    for k in range(_NCH):
        cps[k].wait()
        l1b = l1_v[pl.ds(k * _OC, _OC)]
        uk = u[k * _OC:(k + 1) * _OC]
        vrow = lax.dot_general(uk, l1b, (((2,), (2,)), ((0,), (0,))),
                               preferred_element_type=jnp.float32)
        v_s[pl.ds(k * _OC, _OC), :] = vrow.reshape(_OC, _NI)

    cp_st.wait()
    cp_opt.wait()
    scores = lax.dot_general(state_v[...], v_s[...], (((1,), (1,)), ((), ())),
                             preferred_element_type=jnp.float32)
    onehot = (opt_v[...] == lax.broadcasted_iota(jnp.int32, (1, _NO), 1))
    out_ref[...] = jnp.sum(jnp.where(onehot, scores, 0.0), axis=1,
                           keepdims=True)


def kernel(state, option, action, linear1, linear2, linear3):
    opt = option.astype(jnp.int32).reshape(_B, 1)
    hbm = pl.BlockSpec(memory_space=pltpu.MemorySpace.HBM)
    out = pl.pallas_call(
        _body,
        in_specs=[hbm, hbm, hbm, hbm, hbm],
        out_specs=pl.BlockSpec(memory_space=pltpu.MemorySpace.VMEM),
        out_shape=jax.ShapeDtypeStruct((_B, 1), jnp.float32),
        scratch_shapes=[
            pltpu.VMEM((_NO, _NI, _H), jnp.float32),
            pltpu.VMEM((_NO, _H, _H), jnp.float32),
            pltpu.VMEM((_NO, _H, 1), jnp.float32),
            pltpu.VMEM((_B, _NI), jnp.float32),
            pltpu.VMEM((_B, 1), jnp.int32),
            pltpu.VMEM((_NO, _NI), jnp.float32),
            pltpu.SemaphoreType.DMA((_NCH + 4,)),
        ],
    )(linear1, linear2, linear3, state, opt)
    return out
